# baseline (speedup 1.0000x reference)
import jax
import jax.numpy as jnp
from jax import lax
from jax.experimental import pallas as pl
from jax.experimental.pallas import tpu as pltpu
from jax.experimental.pallas import tpu_sc as plsc

DIM = 1024
TOTAL = 1024 * 50
NUM_WORKERS = 32
PER_WORKER = TOTAL // NUM_WORKERS
CHUNK = 16
NCHUNKS = PER_WORKER // CHUNK
NBUF = 4


def _emb_body(idx_hbm, table_hbm, out_hbm, idx_v,
              buf0, buf1, buf2, buf3, gs0, gs1, gs2, gs3, os0, os1, os2, os3):
    bufs = (buf0, buf1, buf2, buf3)
    gsems = (gs0, gs1, gs2, gs3)
    osems = (os0, os1, os2, os3)
    wid = lax.axis_index("s") * 2 + lax.axis_index("c")
    base = wid * PER_WORKER

    pltpu.sync_copy(idx_hbm.at[wid], idx_v)

    def gather_start(c, b):
        pltpu.make_async_copy(
            table_hbm.at[idx_v.at[c]], bufs[b], gsems[b]
        ).start()

    def gather_wait(b):
        pltpu.make_async_copy(
            table_hbm.at[idx_v.at[0]], bufs[b], gsems[b]
        ).wait()

    def out_start(c, b):
        pltpu.make_async_copy(
            bufs[b], out_hbm.at[pl.ds(base + c * CHUNK, CHUNK)], osems[b]
        ).start()

    def out_wait(c, b):
        pltpu.make_async_copy(
            bufs[b], out_hbm.at[pl.ds(base + c * CHUNK, CHUNK)], osems[b]
        ).wait()

    for b in range(NBUF):
        gather_start(b, b)

    def step(i, carry):
        for b in range(NBUF):
            c = NBUF * i + b
            gather_wait(b)
            out_start(c, b)
            out_wait(c, b)
            gather_start(c + NBUF, b)
        return carry

    lax.fori_loop(0, (NCHUNKS - NBUF) // NBUF, step, 0, unroll=False)

    for b in range(NBUF):
        c = NCHUNKS - NBUF + b
        gather_wait(b)
        out_start(c, b)
        out_wait(c, b)


def kernel(x, emb_weight):
    idx = x.reshape(NUM_WORKERS, NCHUNKS, CHUNK)
    mesh = plsc.VectorSubcoreMesh(core_axis_name="c", subcore_axis_name="s")
    out = pl.kernel(
        _emb_body,
        out_type=jax.ShapeDtypeStruct((TOTAL, DIM), jnp.float32),
        mesh=mesh,
        scratch_types=[
            pltpu.VMEM((NCHUNKS, CHUNK), jnp.int32),
        ] + [pltpu.VMEM((CHUNK, DIM), jnp.float32)] * NBUF
          + [pltpu.SemaphoreType.DMA] * (2 * NBUF),
        compiler_params=pltpu.CompilerParams(use_tc_tiling_on_sc=False),
    )(idx, emb_weight)
    return out.reshape(x.shape[0], x.shape[1], DIM)


# speedup vs baseline: 1.1581x; 1.1581x over previous
import jax
import jax.numpy as jnp
from jax import lax
from jax.experimental import pallas as pl
from jax.experimental.pallas import tpu as pltpu
from jax.experimental.pallas import tpu_sc as plsc

DIM = 1024
TOTAL = 1024 * 50
NUM_WORKERS = 32
PER_WORKER = TOTAL // NUM_WORKERS
CHUNK = 16
NCHUNKS = PER_WORKER // CHUNK
NBUF = 4


def _emb_body(idx_hbm, table_hbm, out_hbm, idx_v,
              buf0, buf1, buf2, buf3, gs0, gs1, gs2, gs3, os0, os1, os2, os3):
    bufs = (buf0, buf1, buf2, buf3)
    gsems = (gs0, gs1, gs2, gs3)
    osems = (os0, os1, os2, os3)
    wid = lax.axis_index("s") * 2 + lax.axis_index("c")
    base = wid * PER_WORKER

    pltpu.sync_copy(idx_hbm.at[wid], idx_v)

    def gather_start(c, b):
        pltpu.make_async_copy(
            table_hbm.at[idx_v.at[c]], bufs[b], gsems[b]
        ).start()

    def gather_wait(b):
        pltpu.make_async_copy(
            table_hbm.at[idx_v.at[0]], bufs[b], gsems[b]
        ).wait()

    def out_start(c, b):
        pltpu.make_async_copy(
            bufs[b], out_hbm.at[pl.ds(base + c * CHUNK, CHUNK)], osems[b]
        ).start()

    def out_wait(c, b):
        pltpu.make_async_copy(
            bufs[b], out_hbm.at[pl.ds(base + c * CHUNK, CHUNK)], osems[b]
        ).wait()

    for b in range(NBUF):
        gather_start(b, b)

    def step(i, carry):
        for b in range(NBUF):
            c = NBUF * i + b
            gather_wait(b)
            out_start(c, b)
            out_wait(c, b)
            gather_start(c + NBUF, b)
        return carry

    lax.fori_loop(0, (NCHUNKS - NBUF) // NBUF, step, 0, unroll=False)

    for b in range(NBUF):
        c = NCHUNKS - NBUF + b
        gather_wait(b)
        out_start(c, b)
        out_wait(c, b)


def kernel(x, emb_weight):
    idx = x.reshape(NUM_WORKERS, NCHUNKS, CHUNK)
    mesh = plsc.VectorSubcoreMesh(core_axis_name="c", subcore_axis_name="s")
    out = pl.kernel(
        _emb_body,
        out_type=jax.ShapeDtypeStruct((TOTAL, DIM), jnp.float32),
        mesh=mesh,
        scratch_types=[
            pltpu.VMEM((NCHUNKS, CHUNK), jnp.int32),
        ] + [pltpu.VMEM((CHUNK, DIM), jnp.float32)] * NBUF
          + [pltpu.SemaphoreType.DMA] * (2 * NBUF),
    )(idx, emb_weight)
    return out.reshape(x.shape[0], x.shape[1], DIM)


# speedup vs baseline: 3.3076x; 2.8559x over previous
import jax
import jax.numpy as jnp
from jax import lax
from jax.experimental import pallas as pl
from jax.experimental.pallas import tpu as pltpu
from jax.experimental.pallas import tpu_sc as plsc

DIM = 1024
BATCH = 1024
SEQ = 50
TOTAL = BATCH * SEQ
NUM_WORKERS = 32
PER_WORKER = TOTAL // NUM_WORKERS
CHUNK = 40
NCHUNKS = PER_WORKER // CHUNK
NBUF = 2


def _emb_body(idx_hbm, table_hbm, out_hbm, idx_v, buf0, buf1,
              gs0, gs1, os0, os1):
    bufs = (buf0, buf1)
    gsems = (gs0, gs1)
    osems = (os0, os1)
    wid = lax.axis_index("s") * 2 + lax.axis_index("c")
    base = wid * PER_WORKER

    pltpu.sync_copy(idx_hbm.at[wid], idx_v)

    def gather_start(c, b):
        pltpu.make_async_copy(
            table_hbm.at[idx_v.at[c]], bufs[b], gsems[b]
        ).start()

    def gather_wait(b):
        pltpu.make_async_copy(
            table_hbm.at[idx_v.at[0]], bufs[b], gsems[b]
        ).wait()

    def out_start(c, b):
        pltpu.make_async_copy(
            bufs[b], out_hbm.at[pl.ds(base + c * CHUNK, CHUNK)], osems[b]
        ).start()

    def out_wait(c, b):
        pltpu.make_async_copy(
            bufs[b], out_hbm.at[pl.ds(base + c * CHUNK, CHUNK)], osems[b]
        ).wait()

    for b in range(NBUF):
        gather_start(b, b)

    def step(i, carry):
        for b in range(NBUF):
            c = NBUF * i + b
            gather_wait(b)
            out_start(c, b)
            out_wait(c, b)
            gather_start(c + NBUF, b)
        return carry

    lax.fori_loop(0, (NCHUNKS - NBUF) // NBUF, step, 0, unroll=False)

    for b in range(NBUF):
        c = NCHUNKS - NBUF + b
        gather_wait(b)
        out_start(c, b)
        out_wait(c, b)


def kernel(x, emb_weight):
    idx = x.T.reshape(NUM_WORKERS, NCHUNKS, CHUNK)
    mesh = plsc.VectorSubcoreMesh(core_axis_name="c", subcore_axis_name="s")
    out = pl.kernel(
        _emb_body,
        out_type=jax.ShapeDtypeStruct((TOTAL, DIM), jnp.float32),
        mesh=mesh,
        scratch_types=[
            pltpu.VMEM((NCHUNKS, CHUNK), jnp.int32),
            pltpu.VMEM((CHUNK, DIM), jnp.float32),
            pltpu.VMEM((CHUNK, DIM), jnp.float32),
            pltpu.SemaphoreType.DMA,
            pltpu.SemaphoreType.DMA,
            pltpu.SemaphoreType.DMA,
            pltpu.SemaphoreType.DMA,
        ],
    )(idx, emb_weight)
    return out.reshape(SEQ, BATCH, DIM).transpose(1, 0, 2)


# speedup vs baseline: 3.3748x; 1.0203x over previous
import jax
import jax.numpy as jnp
from jax import lax
from jax.experimental import pallas as pl
from jax.experimental.pallas import tpu as pltpu
from jax.experimental.pallas import tpu_sc as plsc

DIM = 1024
BATCH = 1024
SEQ = 50
TOTAL = BATCH * SEQ
NUM_WORKERS = 32
PER_WORKER = TOTAL // NUM_WORKERS
CHUNK = 40
NCHUNKS = PER_WORKER // CHUNK
NBUF = 2


def _emb_body(idx_hbm, table_hbm, out_hbm, idx_v, buf0, buf1, smem_stage,
              gs0, gs1, os0, os1):
    bufs = (buf0, buf1)
    sid = lax.axis_index("s")
    gsems = (gs0, gs1)
    osems = (os0, os1)
    wid = lax.axis_index("s") * 2 + lax.axis_index("c")
    base = wid * PER_WORKER

    pltpu.sync_copy(idx_hbm.at[wid], idx_v)

    def gather_start(c, b):
        pltpu.make_async_copy(
            table_hbm.at[idx_v.at[c]], bufs[b], gsems[b]
        ).start()

    def gather_wait(b):
        pltpu.make_async_copy(
            table_hbm.at[idx_v.at[0]], bufs[b], gsems[b]
        ).wait()

    def out_start(c, b):
        pltpu.sync_copy(bufs[b], smem_stage.at[sid])
        pltpu.make_async_copy(
            smem_stage.at[sid], out_hbm.at[pl.ds(base + c * CHUNK, CHUNK)],
            osems[b]
        ).start()

    def out_wait(c, b):
        pltpu.make_async_copy(
            smem_stage.at[sid], out_hbm.at[pl.ds(base + c * CHUNK, CHUNK)],
            osems[b]
        ).wait()

    for b in range(NBUF):
        gather_start(b, b)

    def step(i, carry):
        for b in range(NBUF):
            c = NBUF * i + b
            gather_wait(b)
            out_start(c, b)
            out_wait(c, b)
            gather_start(c + NBUF, b)
        return carry

    lax.fori_loop(0, (NCHUNKS - NBUF) // NBUF, step, 0, unroll=False)

    for b in range(NBUF):
        c = NCHUNKS - NBUF + b
        gather_wait(b)
        out_start(c, b)
        out_wait(c, b)


def kernel(x, emb_weight):
    idx = x.T.reshape(NUM_WORKERS, NCHUNKS, CHUNK)
    mesh = plsc.VectorSubcoreMesh(core_axis_name="c", subcore_axis_name="s")
    out = pl.kernel(
        _emb_body,
        out_type=jax.ShapeDtypeStruct((TOTAL, DIM), jnp.float32),
        mesh=mesh,
        scratch_types=[
            pltpu.VMEM((NCHUNKS, CHUNK), jnp.int32),
            pltpu.VMEM((CHUNK, DIM), jnp.float32),
            pltpu.VMEM((CHUNK, DIM), jnp.float32),
            pltpu.VMEM_SHARED((16, CHUNK, DIM), jnp.float32),
            pltpu.SemaphoreType.DMA,
            pltpu.SemaphoreType.DMA,
            pltpu.SemaphoreType.DMA,
            pltpu.SemaphoreType.DMA,
        ],
    )(idx, emb_weight)
    return out.reshape(SEQ, BATCH, DIM).transpose(1, 0, 2)
